# Initial kernel scaffold; baseline (speedup 1.0000x reference)
import jax
import jax.numpy as jnp
from jax.experimental import pallas as pl


def kernel(entity_embed, W_aux, b_aux, type_tab, W1_0, b1_0, W2_0, b2_0, W1_1, b1_1, W2_1, b2_1, edge_weight, aux_info_all, edge_index, user_ids, item_ids):
    raise NotImplementedError("write your pallas kernel here")


# speedup vs baseline: 2.7443x; 2.7443x over previous
import functools

import jax
import jax.numpy as jnp
from jax import lax
from jax.experimental import pallas as pl
from jax.experimental.pallas import tpu as pltpu
from jax.experimental.pallas import tpu_sc as plsc

_N = 50000
_E = 800000
_OFFSET = 10000
_NC = 2
_NS = 16
_CH = 128
_EP = 800768
_RB = 5000
_ZR = 200
_NZCH = _N // _ZR


def _make_segsum(Dh):
  ET = _EP // _NS
  NITER = ET // _CH
  mesh = plsc.VectorSubcoreMesh(core_axis_name="c", subcore_axis_name="s")

  @functools.partial(
      pl.kernel,
      out_type=jax.ShapeDtypeStruct((_NC, _N, Dh), jnp.float32),
      mesh=mesh,
      compiler_params=pltpu.CompilerParams(needs_layout_passes=False,
                                           use_tc_tiling_on_sc=False),
      scratch_types=[
          pltpu.VMEM((_CH,), jnp.int32),
          pltpu.VMEM((1, _CH), jnp.int32),
          pltpu.VMEM((_CH,), jnp.float32),
          pltpu.VMEM((_CH, Dh), jnp.float32),
          pltpu.VMEM((_ZR, Dh), jnp.float32),
          pltpu.VMEM_SHARED((_N, Dh), jnp.float32),
          pltpu.SemaphoreType.DMA,
      ],
  )
  def seg(tab_hbm, col_hbm, row_hbm, w_hbm, z_hbm, out_hbm,
          idx_v, row_v, w_v, msg_v, zero_v, acc, sem):
    c = lax.axis_index("c")
    s = lax.axis_index("s")
    pltpu.sync_copy(z_hbm, zero_v)
    for k in range((_NZCH + _NS - 1) // _NS):
      cid = k * _NS + s

      @pl.when(cid < _NZCH)
      def _():
        pltpu.sync_copy(zero_v, acc.at[pl.ds(cid * _ZR, _ZR)])

    plsc.subcore_barrier()

    ebase = s * ET
    cN = c * _N

    def body(it, carry):
      e0 = ebase + it * _CH
      pltpu.sync_copy(col_hbm.at[pl.ds(e0, _CH)], idx_v)
      pltpu.sync_copy(row_hbm.at[pl.ds(e0, _CH)], row_v.at[0])
      pltpu.sync_copy(w_hbm.at[pl.ds(e0, _CH)], w_v)
      for g in range(_CH // 16):
        idx_v[pl.ds(g * 16, 16)] = idx_v[pl.ds(g * 16, 16)] + cN
      pltpu.async_copy(tab_hbm.at[idx_v], msg_v, sem).wait()
      for e in range(_CH):
        wsp = plsc.load_gather(w_v, [jnp.full((16,), e, jnp.int32)])
        for d0 in range(Dh // 16):
          msg_v[e, pl.ds(d0 * 16, 16)] = msg_v[e, pl.ds(d0 * 16, 16)] * wsp
      pltpu.sync_copy(msg_v, acc.at[row_v.at[0]], add=True)
      return carry

    lax.fori_loop(0, NITER, body, 0)
    plsc.subcore_barrier()
    for k in range((_NZCH + _NS - 1) // _NS):
      cid = k * _NS + s

      @pl.when(cid < _NZCH)
      def _():
        pltpu.sync_copy(acc.at[pl.ds(cid * _ZR, _ZR)],
                        out_hbm.at[c, pl.ds(cid * _ZR, _ZR)])

  return seg


_make_segsum = functools.cache(_make_segsum)


def _make_gather():
  B = 1024
  RPW = B // (_NC * _NS)
  mesh = plsc.VectorSubcoreMesh(core_axis_name="c", subcore_axis_name="s")
  outs = tuple(jax.ShapeDtypeStruct((B, d), jnp.float32)
               for d in (32, 32, 32, 16)) * 2

  @functools.partial(
      pl.kernel,
      out_type=outs,
      mesh=mesh,
      compiler_params=pltpu.CompilerParams(needs_layout_passes=False,
                                           use_tc_tiling_on_sc=False),
      scratch_types=[
          pltpu.VMEM((RPW,), jnp.int32),
          pltpu.VMEM((RPW,), jnp.int32),
          pltpu.VMEM((RPW, 32), jnp.float32),
          pltpu.VMEM((RPW, 32), jnp.float32),
          pltpu.VMEM((RPW, 32), jnp.float32),
          pltpu.VMEM((RPW, 16), jnp.float32),
          pltpu.SemaphoreType.DMA,
      ],
  )
  def gk(e0_hbm, n1_hbm, n2_hbm, uid_hbm, iid_hbm,
         u0l, u0h, u1, u2, i0l, i0h, i1, i2,
         ids_v, idsn_v, buf_a, buf_b, buf_c, buf_d, sem):
    c = lax.axis_index("c")
    s = lax.axis_index("s")
    base = (s * _NC + c) * RPW
    for id_hbm, o0l, o0h, o1, o2 in ((uid_hbm, u0l, u0h, u1, u2),
                                     (iid_hbm, i0l, i0h, i1, i2)):
      pltpu.sync_copy(id_hbm.at[pl.ds(base, RPW)], ids_v)
      for g in range(RPW // 16):
        idsn_v[pl.ds(g * 16, 16)] = ids_v[pl.ds(g * 16, 16)] + _N
      pltpu.async_copy(e0_hbm.at[ids_v], buf_a, sem).wait()
      pltpu.async_copy(e0_hbm.at[idsn_v], buf_b, sem).wait()
      pltpu.async_copy(n1_hbm.at[ids_v], buf_c, sem).wait()
      pltpu.async_copy(n2_hbm.at[ids_v], buf_d, sem).wait()
      pltpu.sync_copy(buf_a, o0l.at[pl.ds(base, RPW)])
      pltpu.sync_copy(buf_b, o0h.at[pl.ds(base, RPW)])
      pltpu.sync_copy(buf_c, o1.at[pl.ds(base, RPW)])
      pltpu.sync_copy(buf_d, o2.at[pl.ds(base, RPW)])

  return gk


_make_gather = functools.cache(_make_gather)


def _pre_kernel(ent_ref, aux_ref, wt_ref, b_ref, tt_ref, out_ref):
  b = pl.program_id(0)
  x = ent_ref[...]
  a = aux_ref[...]
  gate = jax.nn.sigmoid(jnp.log1p(a) @ wt_ref[...] + b_ref[...]) * 0.15 + 1.0
  f = x * gate
  n = jnp.sqrt(jnp.sum(f * f, axis=1, keepdims=True))
  f = f / jnp.maximum(n, 1e-12)
  rid = b * _RB + lax.broadcasted_iota(jnp.int32, (_RB, 1), 0)
  ego = f + jnp.where(rid >= _OFFSET, tt_ref[1:2, :], tt_ref[0:1, :])
  out_ref[0] = ego[:, :32]
  out_ref[1] = ego[:, 32:]


def _preprocess(entity_embed, aux_p, wt_p, b_aux, type_tab):
  return pl.pallas_call(
      _pre_kernel,
      grid=(_N // _RB,),
      in_specs=[
          pl.BlockSpec((_RB, 64), lambda b: (b, 0)),
          pl.BlockSpec((_RB, 8), lambda b: (b, 0)),
          pl.BlockSpec((8, 64), lambda b: (0, 0)),
          pl.BlockSpec((1, 64), lambda b: (0, 0)),
          pl.BlockSpec((2, 64), lambda b: (0, 0)),
      ],
      out_specs=pl.BlockSpec((2, _RB, 32), lambda b: (0, b, 0)),
      out_shape=jax.ShapeDtypeStruct((2, _N, 32), jnp.float32),
  )(entity_embed, aux_p, wt_p, b_aux, type_tab)


def _layer_body(Do, ego_ref, side_ref, w1_ref, b1_ref, w2_ref, b2_ref,
                outh_ref, norm_ref):
  e = jnp.concatenate([ego_ref[0], ego_ref[1]], axis=1)
  sd = jnp.concatenate([side_ref[0], side_ref[1]], axis=1)
  h1 = (e + sd) @ w1_ref[...] + b1_ref[...]
  h1 = jnp.where(h1 >= 0, h1, 0.01 * h1)
  h2 = (e * sd) @ w2_ref[...] + b2_ref[...]
  h2 = jnp.where(h2 >= 0, h2, 0.01 * h2)
  y = h1 + h2
  outh_ref[0] = y[:, :Do // 2]
  outh_ref[1] = y[:, Do // 2:]
  n = jnp.sqrt(jnp.sum(y * y, axis=1, keepdims=True))
  norm_ref[...] = y / jnp.maximum(n, 1e-12)


def _dense_layer(ego_h, side_h, w1t, b1, w2t, b2):
  Dh = ego_h.shape[-1]
  D = 2 * Dh
  Do = w1t.shape[-1]
  return pl.pallas_call(
      functools.partial(_layer_body, Do),
      grid=(_N // _RB,),
      in_specs=[
          pl.BlockSpec((2, _RB, Dh), lambda b: (0, b, 0)),
          pl.BlockSpec((2, _RB, Dh), lambda b: (0, b, 0)),
          pl.BlockSpec((D, Do), lambda b: (0, 0)),
          pl.BlockSpec((1, Do), lambda b: (0, 0)),
          pl.BlockSpec((D, Do), lambda b: (0, 0)),
          pl.BlockSpec((1, Do), lambda b: (0, 0)),
      ],
      out_specs=[
          pl.BlockSpec((2, _RB, Do // 2), lambda b: (0, b, 0)),
          pl.BlockSpec((_RB, Do), lambda b: (b, 0)),
      ],
      out_shape=[
          jax.ShapeDtypeStruct((2, _N, Do // 2), jnp.float32),
          jax.ShapeDtypeStruct((_N, Do), jnp.float32),
      ],
  )(ego_h, side_h, w1t, b1, w2t, b2)


def _score_kernel(u0l, u0h, u1, u2, i0l, i0h, i1, i2, out_ref):
  dn = (((1,), (1,)), ((), ()))
  acc = lax.dot_general(u0l[...], i0l[...], dn,
                        preferred_element_type=jnp.float32)
  acc = acc + lax.dot_general(u0h[...], i0h[...], dn,
                              preferred_element_type=jnp.float32)
  acc = acc + lax.dot_general(u1[...], i1[...], dn,
                              preferred_element_type=jnp.float32)
  acc = acc + lax.dot_general(u2[...], i2[...], dn,
                              preferred_element_type=jnp.float32)
  out_ref[...] = acc


def _score(parts):
  return pl.pallas_call(
      _score_kernel,
      out_shape=jax.ShapeDtypeStruct((1024, 1024), jnp.float32),
  )(*parts)


def kernel(entity_embed, W_aux, b_aux, type_tab,
           W1_0, b1_0, W2_0, b2_0, W1_1, b1_1, W2_1, b2_1,
           edge_weight, aux_info_all, edge_index, user_ids, item_ids):
  f32 = jnp.float32
  row = edge_index[0]
  col = edge_index[1]
  pad = _EP - _E
  col_p = jnp.concatenate([col, jnp.zeros((pad,), col.dtype)])
  row_p = jnp.concatenate([row, jnp.zeros((pad,), row.dtype)])
  w_p = jnp.concatenate([edge_weight, jnp.zeros((pad,), f32)])
  aux_p = jnp.pad(aux_info_all, ((0, 0), (0, 5)))
  wt_p = jnp.pad(W_aux.T, ((0, 5), (0, 0)))
  z32 = jnp.zeros((_ZR, 32), f32)
  z16 = jnp.zeros((_ZR, 16), f32)

  ego0_h = _preprocess(entity_embed, aux_p, wt_p, b_aux[None, :], type_tab)
  tab0 = ego0_h.reshape(2 * _N, 32)
  side0_h = _make_segsum(32)(tab0, col_p, row_p, w_p, z32)
  ego1_h, n1 = _dense_layer(ego0_h, side0_h, W1_0.T, b1_0[None, :],
                            W2_0.T, b2_0[None, :])
  tab1 = ego1_h.reshape(2 * _N, 16)
  side1_h = _make_segsum(16)(tab1, col_p, row_p, w_p, z16)
  _, n2 = _dense_layer(ego1_h, side1_h, W1_1.T, b1_1[None, :],
                       W2_1.T, b2_1[None, :])
  parts = _make_gather()(tab0, n1, n2, user_ids, item_ids)
  return _score(parts)
